# baseline (speedup 1.0000x reference)
import functools

import jax
import jax.numpy as jnp
from jax import lax
from jax.experimental import pallas as pl
from jax.experimental.pallas import tpu as pltpu
from jax.experimental.pallas import tpu_sc as plsc

NC = 2
NS = 16
NW = NC * NS
B = 8192
H = 128
GX = 512
GYD = 512
CELLS = GX * GYD
SEG = CELLS // NW
PTS = B // NW
TROWS = CELLS + B
K = 25
NCH = PTS * K // 128

_mesh = plsc.VectorSubcoreMesh(core_axis_name="c", subcore_axis_name="s")
_params = pltpu.CompilerParams(needs_layout_passes=False)


def _wid():
    return lax.axis_index("s") * NC + lax.axis_index("c")


@functools.partial(
    pl.kernel,
    out_type=(
        jax.ShapeDtypeStruct((TROWS, H), jnp.float32),
        jax.ShapeDtypeStruct((CELLS,), jnp.int32),
    ),
    mesh=_mesh,
    compiler_params=_params,
    scratch_types=[
        pltpu.VMEM((B,), jnp.int32),
        pltpu.VMEM((B,), jnp.int32),
        pltpu.VMEM((SEG,), jnp.int32),
        pltpu.SemaphoreType.DMA,
        pltpu.SemaphoreType.DMA,
    ],
)
def _build(gx_hbm, gy_hbm, upd_hbm, mem_hbm, table_hbm, owner_hbm,
           gxv, gyv, ownv, sem1, sem2):
    wid = _wid()
    seg0 = wid * SEG
    cp_mem = pltpu.async_copy(
        mem_hbm.at[pl.ds(seg0, SEG)], table_hbm.at[pl.ds(seg0, SEG)], sem1)
    u0 = wid * PTS
    cp_upd = pltpu.async_copy(
        upd_hbm.at[pl.ds(u0, PTS)], table_hbm.at[pl.ds(CELLS + u0, PTS)], sem2)

    pltpu.sync_copy(gx_hbm, gxv)
    pltpu.sync_copy(gy_hbm, gyv)

    neg1 = jnp.full((16,), -1, jnp.int32)

    def init_body(i, carry):
        ownv[pl.ds(i * 16, 16)] = neg1
        return carry

    lax.fori_loop(0, SEG // 16, init_body, 0)

    iota = lax.iota(jnp.int32, 16)

    def scan_body(v, carry):
        b0 = v * 16
        gxc = gxv[pl.ds(b0, 16)]
        gyc = gyv[pl.ds(b0, 16)]
        flat = gxc * GYD + gyc
        _, last = plsc.scan_count(flat)
        local = flat - seg0
        inr = (local >= 0) & (local < SEG)
        lc = jnp.clip(local, 0, SEG - 1)
        plsc.store_scatter(ownv, [lc], b0 + iota, mask=last & inr)
        return carry

    lax.fori_loop(0, B // 16, scan_body, 0)

    pltpu.sync_copy(ownv, owner_hbm.at[pl.ds(seg0, SEG)])
    cp_mem.wait()
    cp_upd.wait()


@functools.partial(
    pl.kernel,
    out_type=jax.ShapeDtypeStruct((B * K, H), jnp.float32),
    mesh=_mesh,
    scratch_types=[
        pltpu.VMEM((PTS,), jnp.int32),
        pltpu.VMEM((PTS,), jnp.int32),
        pltpu.VMEM((NCH, 128), jnp.int32),
        pltpu.VMEM((128,), jnp.int32),
        pltpu.VMEM((128,), jnp.int32),
        pltpu.VMEM((128, H), jnp.float32),
        pltpu.SemaphoreType.DMA,
    ],
)
def _gather(gx_hbm, gy_hbm, owner_hbm, table_hbm, out_hbm,
            gxv, gyv, cellidx, ownb, srcidx, rows, sem):
    wid = _wid()
    p0 = wid * PTS
    pltpu.sync_copy(gx_hbm.at[pl.ds(p0, PTS)], gxv)
    pltpu.sync_copy(gy_hbm.at[pl.ds(p0, PTS)], gyv)

    for k in range(K):
        di = k // 5 - 2
        dj = k % 5 - 2
        for h in range(2):
            j = 2 * k + h

            def pre_body(v, carry, h=h, j=j, di=di, dj=dj):
                b0 = h * 128 + v * 16
                gxc = gxv[pl.ds(b0, 16)]
                gyc = gyv[pl.ds(b0, 16)]
                cx = jnp.maximum(gxc + di, 0)
                cy = jnp.maximum(gyc + dj, 0)
                cellidx[j, pl.ds(v * 16, 16)] = cx * GYD + cy
                return carry

            lax.fori_loop(0, 8, pre_body, 0)

    def chunk_body(j, carry):
        pltpu.async_copy(owner_hbm.at[cellidx.at[j]], ownb, sem).wait()

        def fix_body(u, c2):
            o = ownb[pl.ds(u * 16, 16)]
            cell = cellidx[j, pl.ds(u * 16, 16)]
            srcidx[pl.ds(u * 16, 16)] = jnp.where(o >= 0, o + CELLS, cell)
            return c2

        lax.fori_loop(0, 8, fix_body, 0)
        pltpu.async_copy(table_hbm.at[srcidx], rows, sem).wait()
        out0 = (j // 2) * B + p0 + (j % 2) * 128
        pltpu.sync_copy(rows, out_hbm.at[pl.ds(out0, 128)])
        return carry

    lax.fori_loop(0, NCH, chunk_body, 0)


def kernel(grid_input, updates, memory):
    gx = grid_input[:, 0]
    gy = grid_input[:, 1]
    memflat = memory.reshape(CELLS, H)
    table, owner = _build(gx, gy, updates, memflat)
    outflat = _gather(gx, gy, owner, table)
    return outflat.reshape(B, K, H)


# speedup vs baseline: 9.5693x; 9.5693x over previous
import functools

import jax
import jax.numpy as jnp
from jax import lax
from jax.experimental import pallas as pl
from jax.experimental.pallas import tpu as pltpu
from jax.experimental.pallas import tpu_sc as plsc

NC = 2
NS = 16
NW = NC * NS
B = 8192
H = 128
GYD = 512
CELLS = 512 * 512
SEG = CELLS // NW
PTS = B // NW
K = 25
NCH = PTS * K // 128

_mesh = plsc.VectorSubcoreMesh(core_axis_name="c", subcore_axis_name="s")
_params = pltpu.CompilerParams(needs_layout_passes=False)


def _wid():
    return lax.axis_index("s") * NC + lax.axis_index("c")


@functools.partial(
    pl.kernel,
    out_type=jax.ShapeDtypeStruct((CELLS,), jnp.int32),
    mesh=_mesh,
    compiler_params=_params,
    scratch_types=[
        pltpu.VMEM((B,), jnp.int32),
        pltpu.VMEM((B,), jnp.int32),
        pltpu.VMEM((SEG,), jnp.int32),
    ],
)
def _build(gx_hbm, gy_hbm, owner_hbm, gxv, gyv, ownv):
    wid = _wid()
    seg0 = wid * SEG

    pltpu.sync_copy(gx_hbm, gxv)
    pltpu.sync_copy(gy_hbm, gyv)

    neg1 = jnp.full((16,), -1, jnp.int32)

    def init_body(i, carry):
        ownv[pl.ds(i * 16, 16)] = neg1
        return carry

    lax.fori_loop(0, SEG // 16, init_body, 0)

    iota = lax.iota(jnp.int32, 16)

    def scan_body(v, carry):
        b0 = v * 16
        gxc = gxv[pl.ds(b0, 16)]
        gyc = gyv[pl.ds(b0, 16)]
        flat = gxc * GYD + gyc
        _, last = plsc.scan_count(flat)
        local = flat - seg0
        inr = (local >= 0) & (local < SEG)
        lc = jnp.clip(local, 0, SEG - 1)
        plsc.store_scatter(ownv, [lc], b0 + iota, mask=last & inr)
        return carry

    lax.fori_loop(0, B // 16, scan_body, 0)

    pltpu.sync_copy(ownv, owner_hbm.at[pl.ds(seg0, SEG)])


@functools.partial(
    pl.kernel,
    out_type=jax.ShapeDtypeStruct((B * K, H), jnp.float32),
    mesh=_mesh,
    compiler_params=_params,
    scratch_types=[
        pltpu.VMEM((PTS,), jnp.int32),
        pltpu.VMEM((PTS,), jnp.int32),
        pltpu.VMEM((NCH, 128), jnp.int32),
        pltpu.VMEM((128,), jnp.int32),
        pltpu.VMEM((144,), jnp.int32),
        pltpu.VMEM((144,), jnp.int32),
        pltpu.VMEM((128, H), jnp.float32),
        pltpu.VMEM((16, H), jnp.float32),
        pltpu.SemaphoreType.DMA,
        pltpu.SemaphoreType.DMA,
    ],
)
def _gather(gx_hbm, gy_hbm, owner_hbm, mem_hbm, upd_hbm, out_hbm,
            gxv, gyv, cellidx, ownb, ppos, pown, rows, ubuf, sem, sem2):
    wid = _wid()
    p0 = wid * PTS
    pltpu.sync_copy(gx_hbm.at[pl.ds(p0, PTS)], gxv)
    pltpu.sync_copy(gy_hbm.at[pl.ds(p0, PTS)], gyv)

    iota = lax.iota(jnp.int32, 16)

    for k in range(K):
        di = k // 5 - 2
        dj = k % 5 - 2
        for h in range(2):
            j = 2 * k + h

            def pre_body(v, carry, h=h, j=j, di=di, dj=dj):
                b0 = h * 128 + v * 16
                gxc = gxv[pl.ds(b0, 16)]
                gyc = gyv[pl.ds(b0, 16)]
                cx = jnp.maximum(gxc + di, 0)
                cy = jnp.maximum(gyc + dj, 0)
                cellidx[j, pl.ds(v * 16, 16)] = cx * GYD + cy
                return carry

            lax.fori_loop(0, 8, pre_body, 0)

    def chunk_body(j, carry):
        cp_own = pltpu.async_copy(owner_hbm.at[cellidx.at[j]], ownb, sem)
        cp_rows = pltpu.async_copy(mem_hbm.at[cellidx.at[j]], rows, sem2)
        cp_own.wait()

        def cmp_body(u, cnt):
            o = ownb[pl.ds(u * 16, 16)]
            m = o >= 0
            plsc.store_compressed(ppos.at[pl.ds(cnt, 16)], u * 16 + iota, mask=m)
            plsc.store_compressed(pown.at[pl.ds(cnt, 16)], o, mask=m)
            return cnt + plsc.all_reduce_population_count(m)[0]

        n = lax.fori_loop(0, 8, cmp_body, 0)
        cp_rows.wait()

        @pl.when(n > 0)
        def _patch():
            pos0 = ppos[pl.ds(0, 16)][0]
            own0 = pown[pl.ds(0, 16)][0]

            def batch_body(bi, carry2):
                base = bi * 16
                posv = ppos[pl.ds(base, 16)]
                ownv2 = pown[pl.ds(base, 16)]
                valid = (base + iota) < n
                posv = jnp.where(valid, posv, pos0)
                ownv2 = jnp.where(valid, ownv2, own0)
                pltpu.async_copy(upd_hbm.at[ownv2], ubuf, sem).wait()
                for r in range(16):
                    p = posv[r]
                    for u8 in range(8):
                        rows[p, pl.ds(u8 * 16, 16)] = ubuf[r, pl.ds(u8 * 16, 16)]
                return carry2

            lax.fori_loop(0, (n + 15) // 16, batch_body, 0)

        out0 = (j // 2) * B + p0 + (j % 2) * 128
        pltpu.sync_copy(rows, out_hbm.at[pl.ds(out0, 128)])
        return carry

    lax.fori_loop(0, NCH, chunk_body, 0)


def kernel(grid_input, updates, memory):
    gx = grid_input[:, 0]
    gy = grid_input[:, 1]
    memflat = memory.reshape(CELLS, H)
    owner = _build(gx, gy)
    outflat = _gather(gx, gy, owner, memflat, updates)
    return outflat.reshape(B, K, H)


# speedup vs baseline: 10.8367x; 1.1324x over previous
import functools

import jax
import jax.numpy as jnp
from jax import lax
from jax.experimental import pallas as pl
from jax.experimental.pallas import tpu as pltpu
from jax.experimental.pallas import tpu_sc as plsc

NC = 2
NS = 16
NW = NC * NS
B = 8192
H = 128
GYD = 512
CELLS = 512 * 512
SEG = CELLS // NW
PTS = B // NW
K = 25
NCH = PTS * K // 128

_mesh = plsc.VectorSubcoreMesh(core_axis_name="c", subcore_axis_name="s")
_params = pltpu.CompilerParams(needs_layout_passes=False)


def _wid():
    return lax.axis_index("s") * NC + lax.axis_index("c")


@functools.partial(
    pl.kernel,
    out_type=jax.ShapeDtypeStruct((CELLS,), jnp.int32),
    mesh=_mesh,
    compiler_params=_params,
    scratch_types=[
        pltpu.VMEM((2 * B,), jnp.int32),
        pltpu.VMEM((SEG,), jnp.int32),
    ],
)
def _build(gi_hbm, owner_hbm, giv, ownv):
    wid = _wid()
    seg0 = wid * SEG

    pltpu.sync_copy(gi_hbm, giv)

    neg1 = jnp.full((16,), -1, jnp.int32)

    def init_body(i, carry):
        ownv[pl.ds(i * 16, 16)] = neg1
        return carry

    lax.fori_loop(0, SEG // 16, init_body, 0)

    iota = lax.iota(jnp.int32, 16)

    def scan_body(v, carry):
        b0 = v * 16
        pvec = 2 * (b0 + iota)
        gxc = plsc.load_gather(giv, [pvec])
        gyc = plsc.load_gather(giv, [pvec + 1])
        flat = gxc * GYD + gyc
        _, last = plsc.scan_count(flat)
        local = flat - seg0
        inr = (local >= 0) & (local < SEG)
        lc = jnp.clip(local, 0, SEG - 1)
        plsc.store_scatter(ownv, [lc], b0 + iota, mask=last & inr)
        return carry

    lax.fori_loop(0, B // 16, scan_body, 0)

    pltpu.sync_copy(ownv, owner_hbm.at[pl.ds(seg0, SEG)])


@functools.partial(
    pl.kernel,
    out_type=jax.ShapeDtypeStruct((B * K, H), jnp.float32),
    mesh=_mesh,
    compiler_params=_params,
    scratch_types=[
        pltpu.VMEM((2 * PTS,), jnp.int32),
        pltpu.VMEM((NCH, 128), jnp.int32),
        pltpu.VMEM((2, 128), jnp.int32),
        pltpu.VMEM((144,), jnp.int32),
        pltpu.VMEM((144,), jnp.int32),
        pltpu.VMEM((2, 128, H), jnp.float32),
        pltpu.VMEM((16, H), jnp.float32),
        pltpu.SemaphoreType.DMA((2,)),
        pltpu.SemaphoreType.DMA((2,)),
        pltpu.SemaphoreType.DMA((2,)),
        pltpu.SemaphoreType.DMA,
    ],
)
def _gather(gi_hbm, owner_hbm, mem_hbm, upd_hbm, out_hbm,
            giv, cellidx, ownb, ppos, pown, rows, ubuf,
            semO, semR, semW, semU):
    wid = _wid()
    p0 = wid * PTS
    pltpu.sync_copy(gi_hbm.at[pl.ds(2 * p0, 2 * PTS)], giv)

    iota = lax.iota(jnp.int32, 16)

    for k in range(K):
        di = k // 5 - 2
        dj = k % 5 - 2
        for h in range(2):
            j = 2 * k + h

            def pre_body(v, carry, h=h, j=j, di=di, dj=dj):
                b0 = h * 128 + v * 16
                pvec = 2 * (b0 + iota)
                gxc = plsc.load_gather(giv, [pvec])
                gyc = plsc.load_gather(giv, [pvec + 1])
                cx = jnp.maximum(gxc + di, 0)
                cy = jnp.maximum(gyc + dj, 0)
                cellidx[j, pl.ds(v * 16, 16)] = cx * GYD + cy
                return carry

            lax.fori_loop(0, 8, pre_body, 0)

    def issue(j, s):
        pltpu.async_copy(owner_hbm.at[cellidx.at[j]], ownb.at[s], semO.at[s])
        pltpu.async_copy(mem_hbm.at[cellidx.at[j]], rows.at[s], semR.at[s])

    issue(0, 0)

    def chunk_body(j, carry):
        s = j % 2
        ns = 1 - s

        @pl.when(j + 1 < NCH)
        def _prefetch():
            @pl.when(j >= 1)
            def _drain_out():
                pltpu.make_async_copy(
                    rows.at[ns], out_hbm.at[pl.ds(0, 128)], semW.at[ns]).wait()

            issue(j + 1, ns)

        pltpu.make_async_copy(
            owner_hbm.at[pl.ds(0, 128)], ownb.at[s], semO.at[s]).wait()

        def cmp_body(u, cnt):
            o = ownb[s, pl.ds(u * 16, 16)]
            m = o >= 0
            plsc.store_compressed(ppos.at[pl.ds(cnt, 16)], u * 16 + iota, mask=m)
            plsc.store_compressed(pown.at[pl.ds(cnt, 16)], o, mask=m)
            return cnt + plsc.all_reduce_population_count(m)[0]

        n = lax.fori_loop(0, 8, cmp_body, 0)

        pltpu.make_async_copy(
            mem_hbm.at[pl.ds(0, 128)], rows.at[s], semR.at[s]).wait()

        @pl.when(n > 0)
        def _patch():
            pos0 = ppos[pl.ds(0, 16)][0]
            own0 = pown[pl.ds(0, 16)][0]

            def batch_body(bi, carry2):
                base = bi * 16
                posv = ppos[pl.ds(base, 16)]
                ownv2 = pown[pl.ds(base, 16)]
                valid = (base + iota) < n
                posv = jnp.where(valid, posv, pos0)
                ownv2 = jnp.where(valid, ownv2, own0)
                pltpu.async_copy(upd_hbm.at[ownv2], ubuf, semU).wait()
                for r in range(16):
                    p = posv[r]
                    for u8 in range(8):
                        rows[s, p, pl.ds(u8 * 16, 16)] = ubuf[r, pl.ds(u8 * 16, 16)]
                return carry2

            lax.fori_loop(0, (n + 15) // 16, batch_body, 0)

        out0 = (j // 2) * B + p0 + s * 128
        pltpu.async_copy(rows.at[s], out_hbm.at[pl.ds(out0, 128)], semW.at[s])
        return carry

    lax.fori_loop(0, NCH, chunk_body, 0)

    pltpu.make_async_copy(rows.at[0], out_hbm.at[pl.ds(0, 128)], semW.at[0]).wait()
    pltpu.make_async_copy(rows.at[1], out_hbm.at[pl.ds(0, 128)], semW.at[1]).wait()


def kernel(grid_input, updates, memory):
    gi = grid_input.reshape(2 * B)
    memflat = memory.reshape(CELLS, H)
    owner = _build(gi)
    outflat = _gather(gi, owner, memflat, updates)
    return outflat.reshape(B, K, H)
